# Initial kernel scaffold; baseline (speedup 1.0000x reference)
import jax
import jax.numpy as jnp
from jax.experimental import pallas as pl


def kernel(x, edge_index, W, b):
    raise NotImplementedError("write your pallas kernel here")


# speedup vs baseline: 14.3653x; 14.3653x over previous
import functools

import jax
import jax.numpy as jnp
from jax import lax
from jax.experimental import pallas as pl
from jax.experimental.pallas import tpu as pltpu
from jax.experimental.pallas import tpu_sc as plsc

NC = 2
NS = 16
NW = NC * NS
BATCH = 128
GROUP = 16


def _mesh():
    return plsc.VectorSubcoreMesh(core_axis_name="c", subcore_axis_name="s")


def _hist_kernel(n_pad, nb):
    rpt = n_pad // NS

    @functools.partial(
        pl.kernel,
        out_type=jax.ShapeDtypeStruct((NC * n_pad,), jnp.float32),
        mesh=_mesh(),
        scratch_types=[
            pltpu.VMEM((nb, BATCH), jnp.int32),
            pltpu.VMEM((BATCH,), jnp.float32),
            pltpu.VMEM_SHARED((n_pad,), jnp.float32),
        ],
    )
    def hist(dst_hbm, zeros_hbm, out_hbm, idx_v, ones_v, deg_sp):
        cid = lax.axis_index("c")
        sid = lax.axis_index("s")
        wid = cid * NS + sid
        r0 = sid * rpt
        pltpu.sync_copy(zeros_hbm.at[pl.ds(r0, rpt)], deg_sp.at[pl.ds(r0, rpt)])
        for j in range(BATCH // 16):
            ones_v[pl.ds(j * 16, 16)] = jnp.full((16,), 1.0, jnp.float32)
        pltpu.sync_copy(dst_hbm.at[pl.ds(wid * nb, nb)], idx_v)
        plsc.subcore_barrier()

        def body(b, carry):
            pltpu.sync_copy(ones_v, deg_sp.at[idx_v.at[b]], add=True)
            return carry

        lax.fori_loop(0, nb, body, 0)
        plsc.subcore_barrier()
        pltpu.sync_copy(deg_sp.at[pl.ds(r0, rpt)],
                        out_hbm.at[pl.ds(cid * n_pad + r0, rpt)])

    return hist


def _scatter_kernel(n_pad, nb):
    rpt = n_pad // NS

    @functools.partial(
        pl.kernel,
        out_type=jax.ShapeDtypeStruct((NC * n_pad, 128), jnp.float32),
        mesh=_mesh(),
        scratch_types=[
            pltpu.VMEM((GROUP, BATCH), jnp.int32),
            pltpu.VMEM((GROUP, BATCH), jnp.int32),
            pltpu.VMEM((2, BATCH, 128), jnp.float32),
            pltpu.VMEM_SHARED((n_pad, 128), jnp.float32),
            pltpu.SemaphoreType.DMA,
            pltpu.SemaphoreType.DMA,
        ],
    )
    def scat(h_hbm, src_hbm, dst_hbm, zeros_hbm, out_hbm,
             sidx_v, didx_v, rows_v, acc_sp, sem0, sem1):
        cid = lax.axis_index("c")
        sid = lax.axis_index("s")
        wid = cid * NS + sid
        r0 = sid * rpt
        sems = [sem0, sem1]
        pltpu.sync_copy(zeros_hbm.at[pl.ds(r0, rpt)], acc_sp.at[pl.ds(r0, rpt)])
        plsc.subcore_barrier()

        def group_body(g, carry):
            base = wid * nb + g * GROUP
            pltpu.sync_copy(src_hbm.at[pl.ds(base, GROUP)], sidx_v)
            pltpu.sync_copy(dst_hbm.at[pl.ds(base, GROUP)], didx_v)
            for k in range(2):
                pltpu.async_copy(h_hbm.at[sidx_v.at[k]], rows_v.at[k], sems[k])

            def pair(i, c):
                for k in range(2):
                    j = i * 2 + k
                    pltpu.make_async_copy(
                        h_hbm.at[sidx_v.at[j]], rows_v.at[k], sems[k]).wait()
                    pltpu.sync_copy(rows_v.at[k], acc_sp.at[didx_v.at[j]],
                                    add=True)

                    @pl.when(j + 2 < GROUP)
                    def _():
                        pltpu.async_copy(
                            h_hbm.at[sidx_v.at[j + 2]], rows_v.at[k], sems[k])
                return c

            lax.fori_loop(0, GROUP // 2, pair, 0)
            return carry

        lax.fori_loop(0, nb // GROUP, group_body, 0)
        plsc.subcore_barrier()
        pltpu.sync_copy(acc_sp.at[pl.ds(r0, rpt)],
                        out_hbm.at[pl.ds(cid * n_pad + r0, rpt)])

    return scat


def _matmul_scale(x_pad, w, deg_col, n_pad):
    bm = 1024

    def body(x_ref, w_ref, deg_ref, o_ref):
        dinv = lax.rsqrt(deg_ref[...])
        h = jnp.dot(x_ref[...], w_ref[...], preferred_element_type=jnp.float32)
        o_ref[...] = h * dinv

    return pl.pallas_call(
        body,
        grid=(n_pad // bm,),
        in_specs=[
            pl.BlockSpec((bm, 128), lambda i: (i, 0)),
            pl.BlockSpec((128, 128), lambda i: (0, 0)),
            pl.BlockSpec((bm, 1), lambda i: (i, 0)),
        ],
        out_specs=pl.BlockSpec((bm, 128), lambda i: (i, 0)),
        out_shape=jax.ShapeDtypeStruct((n_pad, 128), jnp.float32),
    )(x_pad, w, deg_col)


def _finalize(acc0, acc1, hprime, deg_col, b_row, n_pad):
    bm = 1024

    def body(a0_ref, a1_ref, h_ref, deg_ref, b_ref, o_ref):
        dinv = lax.rsqrt(deg_ref[...])
        s = a0_ref[...] + a1_ref[...] + h_ref[...]
        o_ref[...] = jnp.maximum(s * dinv + b_ref[...], 0.0)

    return pl.pallas_call(
        body,
        grid=(n_pad // bm,),
        in_specs=[
            pl.BlockSpec((bm, 128), lambda i: (i, 0)),
            pl.BlockSpec((bm, 128), lambda i: (i, 0)),
            pl.BlockSpec((bm, 128), lambda i: (i, 0)),
            pl.BlockSpec((bm, 1), lambda i: (i, 0)),
            pl.BlockSpec((1, 128), lambda i: (0, 0)),
        ],
        out_specs=pl.BlockSpec((bm, 128), lambda i: (i, 0)),
        out_shape=jax.ShapeDtypeStruct((n_pad, 128), jnp.float32),
    )(acc0, acc1, hprime, deg_col, b_row)


def kernel(x, edge_index, W, b):
    n, hidden = x.shape
    e = edge_index.shape[1]
    n_pad = -(-n // (NS * 128)) * (NS * 128)
    nb = -(-e // (NW * BATCH))
    nb = -(-nb // GROUP) * GROUP
    e_pad = NW * nb * BATCH

    src = edge_index[0].astype(jnp.int32)
    dst = edge_index[1].astype(jnp.int32)
    pad_idx = jnp.full((e_pad - e,), n, jnp.int32)
    src2 = jnp.concatenate([src, pad_idx]).reshape(NW * nb, BATCH)
    dst2 = jnp.concatenate([dst, pad_idx]).reshape(NW * nb, BATCH)

    x_pad = jnp.pad(x, ((0, n_pad - n), (0, 0)))
    zeros1 = jnp.zeros((n_pad,), jnp.float32)
    zeros2 = jnp.zeros((n_pad, 128), jnp.float32)

    hist = _hist_kernel(n_pad, nb)(dst2, zeros1)
    deg_col = (1.0 + hist[:n_pad] + hist[n_pad:]).reshape(n_pad, 1)

    hprime = _matmul_scale(x_pad, W, deg_col, n_pad)
    acc = _scatter_kernel(n_pad, nb)(hprime, src2, dst2, zeros2)
    out = _finalize(acc[:n_pad], acc[n_pad:], hprime, deg_col,
                    b.reshape(1, 128), n_pad)
    return out[:n]
